# Initial kernel scaffold; baseline (speedup 1.0000x reference)
import jax
import jax.numpy as jnp
from jax.experimental import pallas as pl


def kernel(x, edge_index, batch, W0, b0, W1, b1, W2, b2, W3, b3, Wr, br):
    raise NotImplementedError("write your pallas kernel here")


# speedup vs baseline: 2.5789x; 2.5789x over previous
import jax
import jax.numpy as jnp
from jax import lax
from jax.experimental import pallas as pl
from jax.experimental.pallas import tpu as pltpu
from jax.experimental.pallas import tpu_sc as plsc

NC = 2
NS = 16
NW = NC * NS

EB = 128
G = 64
CS = 128
NSL = 8
RB = 1000


def _deg_body(dst_hbm, z_hbm, o_hbm, out_hbm, idx_v, ones_v, z_v, dacc,
              *, nb, nbp, nacc, stripe, zch):
    cid = lax.axis_index("c")
    sid = lax.axis_index("s")
    wid = cid * NS + sid
    pltpu.sync_copy(dst_hbm.at[pl.ds(wid * nbp, nbp)], idx_v)
    pltpu.sync_copy(z_hbm, z_v)
    pltpu.sync_copy(o_hbm, ones_v)
    for z in range(zch):
        pltpu.sync_copy(z_v, dacc.at[pl.ds(sid * stripe + z * 128, 128)])
    plsc.subcore_barrier()
    for b in range(nb):
        pltpu.sync_copy(ones_v, dacc.at[idx_v.at[b]], add=True)
    plsc.subcore_barrier()
    for z in range(zch):
        pltpu.sync_copy(dacc.at[pl.ds(sid * stripe + z * 128, 128)], z_v)
        pltpu.sync_copy(
            z_v, out_hbm.at[pl.ds(cid * nacc + sid * stripe + z * 128, 128)])


def _scat_body(y_hbm, src8_hbm, dst_hbm, z_hbm, out_hbm,
               src_v, dst_v, z_v, rows_v, acc,
               *, nb, nbp, nacc, stripe, zch):
    cid = lax.axis_index("c")
    sid = lax.axis_index("s")
    pltpu.sync_copy(z_hbm, z_v)
    for p in range(NSL // NC):
        sl = NC * p + cid
        for z in range(zch):
            pltpu.sync_copy(z_v, acc.at[pl.ds(sid * stripe + z * 128, 128)])
        plsc.subcore_barrier()
        for q in range(NC):
            chunk = q * NS + sid
            pltpu.sync_copy(dst_hbm.at[pl.ds(chunk * nbp, nbp)], dst_v)
            pltpu.sync_copy(src8_hbm.at[pl.ds((sl * NW + chunk) * nbp, nbp)], src_v)
            for b in range(nb):
                pltpu.sync_copy(y_hbm.at[src_v.at[b]], rows_v)
                pltpu.sync_copy(rows_v, acc.at[dst_v.at[b]], add=True)
        plsc.subcore_barrier()
        for z in range(zch):
            pltpu.sync_copy(acc.at[pl.ds(sid * stripe + z * 128, 128)], rows_v)
            pltpu.sync_copy(
                rows_v, out_hbm.at[pl.ds(sl * nacc + sid * stripe + z * 128, 128)])


def _mm_body(x_ref, dg_ref, w_ref, y_ref, dv_ref):
    dg = dg_ref[...]
    dsum = 1.0 + dg[0, :, 0:1] + dg[1, :, 0:1]
    dv = lax.rsqrt(dsum)
    dv_ref[...] = dv
    ys = jnp.dot(x_ref[...] * dv, w_ref[...], preferred_element_type=jnp.float32)
    for c in range(NSL):
        y_ref[c] = ys[:, c * CS:(c + 1) * CS]


def _pool_body(s_ref, y_ref, dv_ref, p_ref, b3_ref, wr_ref, br_ref,
               out_ref, pool_s, cnt_s, *, nrb):
    r = pl.program_id(0)

    @pl.when(r == 0)
    def _init():
        pool_s[...] = jnp.zeros_like(pool_s)
        cnt_s[...] = jnp.zeros_like(cnt_s)

    dv = dv_ref[...]
    pb = p_ref[...]
    ones = jnp.ones((RB, CS), jnp.float32)
    dn = (((0,), (0,)), ((), ()))
    cnt_s[...] += lax.dot_general(pb, ones, dn, preferred_element_type=jnp.float32)
    for c in range(NSL):
        hc = jnp.maximum(dv * (s_ref[c] + y_ref[c]) + b3_ref[c], 0.0)
        pool_s[c] += lax.dot_general(pb, hc, dn, preferred_element_type=jnp.float32)

    @pl.when(r == nrb - 1)
    def _fin():
        o = jnp.zeros((G, 2), jnp.float32)
        for c in range(NSL):
            o = o + jnp.dot(pool_s[c], wr_ref[c], preferred_element_type=jnp.float32)
        cnt = jnp.maximum(cnt_s[:, 0:1], 1.0)
        out_ref[...] = o / cnt + br_ref[...]


def _sc_degree(dstp, zeros128, ones128, nb, nbp, nacc, stripe, zch):
    import functools
    mesh = plsc.VectorSubcoreMesh(
        core_axis_name="c", subcore_axis_name="s", num_cores=NC, num_subcores=NS)
    deg_fn = pl.kernel(
        functools.partial(_deg_body, nb=nb, nbp=nbp, nacc=nacc, stripe=stripe,
                          zch=zch),
        out_type=jax.ShapeDtypeStruct((NC * nacc, 128), jnp.float32),
        mesh=mesh,
        scratch_types=[
            pltpu.VMEM((nbp, EB), jnp.int32),
            pltpu.VMEM((EB, 128), jnp.float32),
            pltpu.VMEM((128, 128), jnp.float32),
            pltpu.VMEM_SHARED((nacc, 128), jnp.float32),
        ],
    )
    return deg_fn(dstp, zeros128, ones128)


def _sc_scatter(yflat, src8, dstp, zeros128, nb, nbp, nacc, stripe, zch):
    import functools
    mesh = plsc.VectorSubcoreMesh(
        core_axis_name="c", subcore_axis_name="s", num_cores=NC, num_subcores=NS)
    scat_fn = pl.kernel(
        functools.partial(_scat_body, nb=nb, nbp=nbp, nacc=nacc, stripe=stripe,
                          zch=zch),
        out_type=jax.ShapeDtypeStruct((NSL * nacc, CS), jnp.float32),
        mesh=mesh,
        scratch_types=[
            pltpu.VMEM((nbp, EB), jnp.int32),
            pltpu.VMEM((nbp, EB), jnp.int32),
            pltpu.VMEM((128, 128), jnp.float32),
            pltpu.VMEM((EB, CS), jnp.float32),
            pltpu.VMEM_SHARED((nacc, CS), jnp.float32),
        ],
    )
    return scat_fn(yflat, src8, dstp, zeros128)


def kernel(x, edge_index, batch, W0, b0, W1, b1, W2, b2, W3, b3, Wr, br):
    import functools

    N, D = x.shape
    E = edge_index.shape[1]
    nrb = N // RB
    nb = -(-E // (NW * EB))
    nbp = -(-nb // 8) * 8
    epad = NW * nb * EB
    nacc = ((N // (128 * NS)) + 1) * 128 * NS
    stripe = nacc // NS
    zch = stripe // 128

    src = edge_index[0].astype(jnp.int32)
    dst = edge_index[1].astype(jnp.int32)
    pad = epad - E
    dst3 = jnp.concatenate([dst, jnp.full((pad,), N, jnp.int32)]).reshape(NW, nb, EB)
    dstp = jnp.pad(dst3, ((0, 0), (0, nbp - nb), (0, 0))).reshape(NW * nbp, EB)
    src3 = jnp.concatenate([src, jnp.zeros((pad,), jnp.int32)]).reshape(NW, nb, EB)
    src8 = (src3[None] + (jnp.arange(NSL, dtype=jnp.int32) * N)[:, None, None, None])
    src8 = jnp.pad(src8, ((0, 0), (0, 0), (0, nbp - nb), (0, 0)))
    src8 = src8.reshape(NSL * NW * nbp, EB)
    pt = (batch[:, None] == jnp.arange(G, dtype=batch.dtype)[None, :]).astype(jnp.float32)
    zeros128 = jnp.zeros((128, 128), jnp.float32)
    ones128 = jnp.ones((EB, 128), jnp.float32)
    b3r = b3.reshape(NSL, 1, CS)
    wrr = Wr.reshape(NSL, CS, 2)
    brr = br.reshape(1, 2)

    deg2 = _sc_degree(dstp, zeros128, ones128, nb, nbp, nacc,
                      stripe, zch).reshape(NC, nacc, 128)

    y8, dinv = pl.pallas_call(
        _mm_body,
        grid=(nrb,),
        in_specs=[
            pl.BlockSpec((RB, D), lambda r: (r, 0)),
            pl.BlockSpec((NC, RB, 128), lambda r: (0, r, 0)),
            pl.BlockSpec((D, D), lambda r: (0, 0)),
        ],
        out_specs=[
            pl.BlockSpec((NSL, RB, CS), lambda r: (0, r, 0)),
            pl.BlockSpec((RB, 1), lambda r: (r, 0)),
        ],
        out_shape=[
            jax.ShapeDtypeStruct((NSL, N, CS), jnp.float32),
            jax.ShapeDtypeStruct((N, 1), jnp.float32),
        ],
    )(x, deg2, W3)

    s8 = _sc_scatter(y8.reshape(NSL * N, CS), src8, dstp, zeros128,
                     nb, nbp, nacc, stripe, zch).reshape(NSL, nacc, CS)

    out = pl.pallas_call(
        functools.partial(_pool_body, nrb=nrb),
        grid=(nrb,),
        in_specs=[
            pl.BlockSpec((NSL, RB, CS), lambda r: (0, r, 0)),
            pl.BlockSpec((NSL, RB, CS), lambda r: (0, r, 0)),
            pl.BlockSpec((RB, 1), lambda r: (r, 0)),
            pl.BlockSpec((RB, G), lambda r: (r, 0)),
            pl.BlockSpec((NSL, 1, CS), lambda r: (0, 0, 0)),
            pl.BlockSpec((NSL, CS, 2), lambda r: (0, 0, 0)),
            pl.BlockSpec((1, 2), lambda r: (0, 0)),
        ],
        out_specs=pl.BlockSpec((G, 2), lambda r: (0, 0)),
        out_shape=jax.ShapeDtypeStruct((G, 2), jnp.float32),
        scratch_shapes=[
            pltpu.VMEM((NSL, G, CS), jnp.float32),
            pltpu.VMEM((G, CS), jnp.float32),
        ],
    )(s8, y8, dinv, pt, b3r, wrr, brr)
    return out


# speedup vs baseline: 2.7399x; 1.0624x over previous
import jax
import jax.numpy as jnp
from jax import lax
from jax.experimental import pallas as pl
from jax.experimental.pallas import tpu as pltpu
from jax.experimental.pallas import tpu_sc as plsc

NC = 2
NS = 16
NW = NC * NS

EB = 128
G = 64
CS = 128
NSL = 8
RB = 1000


def _deg_body(dst_hbm, z_hbm, o_hbm, out_hbm, idx_v, ones_v, z_v, dacc,
              *, nb, nbp, nacc, stripe, zch):
    cid = lax.axis_index("c")
    sid = lax.axis_index("s")
    wid = cid * NS + sid
    pltpu.sync_copy(dst_hbm.at[pl.ds(wid * nbp, nbp)], idx_v)
    pltpu.sync_copy(z_hbm, z_v)
    pltpu.sync_copy(o_hbm, ones_v)
    for z in range(zch):
        pltpu.sync_copy(z_v, dacc.at[pl.ds(sid * stripe + z * 128, 128)])
    plsc.subcore_barrier()
    for b in range(nb):
        pltpu.sync_copy(ones_v, dacc.at[idx_v.at[b]], add=True)
    plsc.subcore_barrier()
    for z in range(zch):
        pltpu.sync_copy(dacc.at[pl.ds(sid * stripe + z * 128, 128)], z_v)
        pltpu.sync_copy(
            z_v, out_hbm.at[pl.ds(cid * nacc + sid * stripe + z * 128, 128)])


def _scat_body(y_hbm, src8_hbm, dst_hbm, z_hbm, out_hbm,
               src_v, dst_v, rows_a, rows_b,
               acc, sem_ga, sem_gb, sem_sa, sem_sb, sem_x,
               *, nb, nbp, nacc, stripe, zch):
    cid = lax.axis_index("c")
    sid = lax.axis_index("s")
    rows = (rows_a, rows_b)
    semg = (sem_ga, sem_gb)
    sems = (sem_sa, sem_sb)
    nbt = NC * nb
    d0 = pltpu.async_copy(dst_hbm.at[pl.ds(sid * nbp, nbp)],
                          dst_v.at[pl.ds(0, nbp)], sem_sa)
    d1 = pltpu.async_copy(dst_hbm.at[pl.ds((NS + sid) * nbp, nbp)],
                          dst_v.at[pl.ds(nbp, nbp)], sem_sb)
    d0.wait()
    d1.wait()
    for p in range(NSL // NC):
        sl = NC * p + cid
        s0 = pltpu.async_copy(src8_hbm.at[pl.ds((sl * NW + sid) * nbp, nbp)],
                              src_v.at[pl.ds(0, nbp)], sem_ga)
        s1 = pltpu.async_copy(src8_hbm.at[pl.ds((sl * NW + NS + sid) * nbp, nbp)],
                              src_v.at[pl.ds(nbp, nbp)], sem_gb)
        pltpu.sync_copy(z_hbm, rows_a)
        zd = [pltpu.async_copy(rows_a,
                               acc.at[pl.ds(sid * stripe + z * 128, 128)], sem_x)
              for z in range(zch)]
        s0.wait()
        s1.wait()
        for d in zd:
            d.wait()
        plsc.subcore_barrier()

        def gstart(j):
            rj = (j // nb) * nbp + (j % nb)
            return pltpu.async_copy(y_hbm.at[src_v.at[rj]], rows[j % 2],
                                    semg[j % 2])

        def sstart(j):
            rj = (j // nb) * nbp + (j % nb)
            return pltpu.async_copy(rows[j % 2], acc.at[dst_v.at[rj]],
                                    sems[j % 2], add=True)

        g = {0: gstart(0)}
        s = {}
        for j in range(nbt):
            g[j].wait()
            if j + 1 < nbt:
                if j >= 1:
                    s[j - 1].wait()
                g[j + 1] = gstart(j + 1)
            s[j] = sstart(j)
        s[nbt - 2].wait()
        s[nbt - 1].wait()
        plsc.subcore_barrier()

        def co_in(z):
            return pltpu.async_copy(acc.at[pl.ds(sid * stripe + z * 128, 128)],
                                    rows[z % 2], semg[z % 2])

        def co_out(z):
            return pltpu.async_copy(
                rows[z % 2],
                out_hbm.at[pl.ds(sl * nacc + sid * stripe + z * 128, 128)],
                sems[z % 2])

        ci = {0: co_in(0)}
        co = {}
        for z in range(zch):
            ci[z].wait()
            if z + 1 < zch:
                if z >= 1:
                    co[z - 1].wait()
                ci[z + 1] = co_in(z + 1)
            co[z] = co_out(z)
        co[zch - 2].wait()
        co[zch - 1].wait()


def _mm_body(x_ref, dg_ref, w_ref, y_ref, dv_ref):
    dg = dg_ref[...]
    dsum = 1.0 + dg[0, :, 0:1] + dg[1, :, 0:1]
    dv = lax.rsqrt(dsum)
    dv_ref[...] = dv
    ys = jnp.dot(x_ref[...] * dv, w_ref[...], preferred_element_type=jnp.float32)
    for c in range(NSL):
        y_ref[c] = ys[:, c * CS:(c + 1) * CS]


def _pool_body(s_ref, y_ref, dv_ref, p_ref, b3_ref, wr_ref, br_ref,
               out_ref, pool_s, cnt_s, *, nrb):
    r = pl.program_id(0)

    @pl.when(r == 0)
    def _init():
        pool_s[...] = jnp.zeros_like(pool_s)
        cnt_s[...] = jnp.zeros_like(cnt_s)

    dv = dv_ref[...]
    pb = p_ref[...]
    ones = jnp.ones((RB, CS), jnp.float32)
    dn = (((0,), (0,)), ((), ()))
    cnt_s[...] += lax.dot_general(pb, ones, dn, preferred_element_type=jnp.float32)
    for c in range(NSL):
        hc = jnp.maximum(dv * (s_ref[c] + y_ref[c]) + b3_ref[c], 0.0)
        pool_s[c] += lax.dot_general(pb, hc, dn, preferred_element_type=jnp.float32)

    @pl.when(r == nrb - 1)
    def _fin():
        o = jnp.zeros((G, 2), jnp.float32)
        for c in range(NSL):
            o = o + jnp.dot(pool_s[c], wr_ref[c], preferred_element_type=jnp.float32)
        cnt = jnp.maximum(cnt_s[:, 0:1], 1.0)
        out_ref[...] = o / cnt + br_ref[...]


def _sc_degree(dstp, zeros128, ones128, nb, nbp, nacc, stripe, zch):
    import functools
    mesh = plsc.VectorSubcoreMesh(
        core_axis_name="c", subcore_axis_name="s", num_cores=NC, num_subcores=NS)
    deg_fn = pl.kernel(
        functools.partial(_deg_body, nb=nb, nbp=nbp, nacc=nacc, stripe=stripe,
                          zch=zch),
        out_type=jax.ShapeDtypeStruct((NC * nacc, 128), jnp.float32),
        mesh=mesh,
        scratch_types=[
            pltpu.VMEM((nbp, EB), jnp.int32),
            pltpu.VMEM((EB, 128), jnp.float32),
            pltpu.VMEM((128, 128), jnp.float32),
            pltpu.VMEM_SHARED((nacc, 128), jnp.float32),
        ],
    )
    return deg_fn(dstp, zeros128, ones128)


def _sc_scatter(yflat, src8, dstp, zeros128, nb, nbp, nacc, stripe, zch):
    import functools
    mesh = plsc.VectorSubcoreMesh(
        core_axis_name="c", subcore_axis_name="s", num_cores=NC, num_subcores=NS)
    scat_fn = pl.kernel(
        functools.partial(_scat_body, nb=nb, nbp=nbp, nacc=nacc, stripe=stripe,
                          zch=zch),
        out_type=jax.ShapeDtypeStruct((NSL * nacc, CS), jnp.float32),
        mesh=mesh,
        scratch_types=[
            pltpu.VMEM((NC * nbp, EB), jnp.int32),
            pltpu.VMEM((NC * nbp, EB), jnp.int32),
            pltpu.VMEM((EB, CS), jnp.float32),
            pltpu.VMEM((EB, CS), jnp.float32),
            pltpu.VMEM_SHARED((nacc, CS), jnp.float32),
            pltpu.SemaphoreType.DMA,
            pltpu.SemaphoreType.DMA,
            pltpu.SemaphoreType.DMA,
            pltpu.SemaphoreType.DMA,
            pltpu.SemaphoreType.DMA,
        ],
    )
    return scat_fn(yflat, src8, dstp, zeros128)


def kernel(x, edge_index, batch, W0, b0, W1, b1, W2, b2, W3, b3, Wr, br):
    import functools

    N, D = x.shape
    E = edge_index.shape[1]
    nrb = N // RB
    nb = -(-E // (NW * EB))
    nbp = -(-nb // 8) * 8
    epad = NW * nb * EB
    nacc = ((N // (128 * NS)) + 1) * 128 * NS
    stripe = nacc // NS
    zch = stripe // 128

    src = edge_index[0].astype(jnp.int32)
    dst = edge_index[1].astype(jnp.int32)
    pad = epad - E
    dst3 = jnp.concatenate([dst, jnp.full((pad,), N, jnp.int32)]).reshape(NW, nb, EB)
    dstp = jnp.pad(dst3, ((0, 0), (0, nbp - nb), (0, 0))).reshape(NW * nbp, EB)
    src3 = jnp.concatenate([src, jnp.zeros((pad,), jnp.int32)]).reshape(NW, nb, EB)
    src8 = (src3[None] + (jnp.arange(NSL, dtype=jnp.int32) * N)[:, None, None, None])
    src8 = jnp.pad(src8, ((0, 0), (0, 0), (0, nbp - nb), (0, 0)))
    src8 = src8.reshape(NSL * NW * nbp, EB)
    pt = (batch[:, None] == jnp.arange(G, dtype=batch.dtype)[None, :]).astype(jnp.float32)
    zeros128 = jnp.zeros((128, 128), jnp.float32)
    ones128 = jnp.ones((EB, 128), jnp.float32)
    b3r = b3.reshape(NSL, 1, CS)
    wrr = Wr.reshape(NSL, CS, 2)
    brr = br.reshape(1, 2)

    deg2 = _sc_degree(dstp, zeros128, ones128, nb, nbp, nacc,
                      stripe, zch).reshape(NC, nacc, 128)

    y8, dinv = pl.pallas_call(
        _mm_body,
        grid=(nrb,),
        in_specs=[
            pl.BlockSpec((RB, D), lambda r: (r, 0)),
            pl.BlockSpec((NC, RB, 128), lambda r: (0, r, 0)),
            pl.BlockSpec((D, D), lambda r: (0, 0)),
        ],
        out_specs=[
            pl.BlockSpec((NSL, RB, CS), lambda r: (0, r, 0)),
            pl.BlockSpec((RB, 1), lambda r: (r, 0)),
        ],
        out_shape=[
            jax.ShapeDtypeStruct((NSL, N, CS), jnp.float32),
            jax.ShapeDtypeStruct((N, 1), jnp.float32),
        ],
    )(x, deg2, W3)

    s8 = _sc_scatter(y8.reshape(NSL * N, CS), src8, dstp, zeros128,
                     nb, nbp, nacc, stripe, zch).reshape(NSL, nacc, CS)

    out = pl.pallas_call(
        functools.partial(_pool_body, nrb=nrb),
        grid=(nrb,),
        in_specs=[
            pl.BlockSpec((NSL, RB, CS), lambda r: (0, r, 0)),
            pl.BlockSpec((NSL, RB, CS), lambda r: (0, r, 0)),
            pl.BlockSpec((RB, 1), lambda r: (r, 0)),
            pl.BlockSpec((RB, G), lambda r: (r, 0)),
            pl.BlockSpec((NSL, 1, CS), lambda r: (0, 0, 0)),
            pl.BlockSpec((NSL, CS, 2), lambda r: (0, 0, 0)),
            pl.BlockSpec((1, 2), lambda r: (0, 0)),
        ],
        out_specs=pl.BlockSpec((G, 2), lambda r: (0, 0)),
        out_shape=jax.ShapeDtypeStruct((G, 2), jnp.float32),
        scratch_shapes=[
            pltpu.VMEM((NSL, G, CS), jnp.float32),
            pltpu.VMEM((G, CS), jnp.float32),
        ],
    )(s8, y8, dinv, pt, b3r, wrr, brr)
    return out
